# baseline (speedup 1.0000x reference)
import functools

import jax
import jax.numpy as jnp
from jax import lax
from jax.experimental import pallas as pl
from jax.experimental.pallas import tpu as pltpu
from jax.experimental.pallas import tpu_sc as plsc

N = 10000
D_IN = 128
H = 256
C = 47
CPAD = 128

NC = 2
NS = 16
NW = NC * NS
LANES = 16

NPAD = 10240
RPT = NPAD // NW
CHUNK = 128
IDXB = 8
RB = 1280


def _make_sc_agg(d, nchunk, with_deg):
    mesh = plsc.VectorSubcoreMesh(core_axis_name="c", subcore_axis_name="s")
    out_type = [jax.ShapeDtypeStruct((NPAD, d), jnp.float32)]
    if with_deg:
        out_type.append(jax.ShapeDtypeStruct((NPAD, d), jnp.float32))
    npc = NPAD // NC
    scratch = [
        pltpu.VMEM((IDXB, CHUNK), jnp.int32),
        pltpu.VMEM((IDXB, CHUNK), jnp.int32),
        pltpu.VMEM((2 * CHUNK,), jnp.int32),
        pltpu.VMEM((2 * CHUNK,), jnp.int32),
        pltpu.VMEM((1, CHUNK), jnp.int32),
        pltpu.VMEM((1, CHUNK), jnp.int32),
        pltpu.VMEM((CHUNK, d), jnp.float32),
        pltpu.VMEM((IDXB, d), jnp.float32),
        pltpu.VMEM_SHARED((npc + 8, d), jnp.float32),
        pltpu.SemaphoreType.DMA,
    ]
    if with_deg:
        scratch += [
            pltpu.VMEM((CHUNK, d), jnp.float32),
            pltpu.VMEM_SHARED((npc + 8, d), jnp.float32),
        ]

    def body(feat, src2d, dst2d, *rest):
        if with_deg:
            out, dout = rest[0], rest[1]
            (sidx, didx, csrc, cdst, fsrc, fdst, rows, zbuf, acc, sem,
             ones, dgacc) = rest[2:]
        else:
            out = rest[0]
            (sidx, didx, csrc, cdst, fsrc, fdst, rows, zbuf, acc,
             sem) = rest[1:]
        c = lax.axis_index("c")
        s = lax.axis_index("s")
        w = c * NS + s
        lo = w * RPT
        rel = s * RPT

        for i in range(IDXB):
            for j in range(d // LANES):
                zbuf[i, pl.ds(j * LANES, LANES)] = jnp.zeros(
                    (LANES,), jnp.float32)

        def zcp(t, _):
            pltpu.sync_copy(zbuf, acc.at[pl.ds(rel + t * IDXB, IDXB)])
            if with_deg:
                pltpu.sync_copy(zbuf, dgacc.at[pl.ds(rel + t * IDXB, IDXB)])
            return 0
        lax.fori_loop(0, RPT // IDXB, zcp, 0)
        if with_deg:
            def orow(i, _):
                for j in range(d // LANES):
                    ones[i, pl.ds(j * LANES, LANES)] = jnp.ones(
                        (LANES,), jnp.float32)
                return 0
            lax.fori_loop(0, CHUNK, orow, 0)
        for v in range(2 * CHUNK // LANES):
            csrc[pl.ds(v * LANES, LANES)] = jnp.zeros((LANES,), jnp.int32)
            cdst[pl.ds(v * LANES, LANES)] = jnp.zeros((LANES,), jnp.int32)

        def fire():
            for v in range(CHUNK // LANES):
                fsrc[0, pl.ds(v * LANES, LANES)] = csrc[
                    pl.ds(v * LANES, LANES)]
                fdst[0, pl.ds(v * LANES, LANES)] = cdst[
                    pl.ds(v * LANES, LANES)]
            pltpu.async_copy(feat.at[fsrc.at[0]], rows, sem).wait()
            pltpu.sync_copy(rows, acc.at[fdst.at[0]], add=True)
            if with_deg:
                pltpu.sync_copy(ones, dgacc.at[fdst.at[0]], add=True)

        def drain(cnt):
            def do(cnt):
                fire()
                for v in range(CHUNK // LANES):
                    csrc[pl.ds(v * LANES, LANES)] = csrc[
                        pl.ds(CHUNK + v * LANES, LANES)]
                    cdst[pl.ds(v * LANES, LANES)] = cdst[
                        pl.ds(CHUNK + v * LANES, LANES)]
                return cnt - CHUNK
            return lax.cond(cnt >= CHUNK, do, lambda q: q, cnt)

        def blk(t, cnt):
            b = pl.multiple_of(t * IDXB, IDXB)
            pltpu.sync_copy(src2d.at[pl.ds(b, IDXB)], sidx)
            pltpu.sync_copy(dst2d.at[pl.ds(b, IDXB)], didx)
            for g in range(IDXB):
                for u in range(CHUNK // LANES):
                    dv = didx[g, pl.ds(u * LANES, LANES)]
                    sv = sidx[g, pl.ds(u * LANES, LANES)]
                    m = (dv >= lo) & (dv < lo + RPT)
                    plsc.store_compressed(
                        csrc.at[pl.ds(cnt, LANES)], sv, mask=m)
                    plsc.store_compressed(
                        cdst.at[pl.ds(cnt, LANES)], dv - (lo - rel), mask=m)
                    cnt = cnt + jnp.max(
                        plsc.all_reduce_population_count(m))
                cnt = drain(cnt)
            return cnt
        cnt = lax.fori_loop(0, nchunk // IDXB, blk, jnp.int32(0))

        pos = lax.iota(jnp.int32, LANES)
        for v in range(CHUNK // LANES):
            live = (pos + v * LANES) < cnt
            sv = csrc[pl.ds(v * LANES, LANES)]
            dv = cdst[pl.ds(v * LANES, LANES)]
            csrc[pl.ds(v * LANES, LANES)] = jnp.where(live, sv, 0)
            cdst[pl.ds(v * LANES, LANES)] = jnp.where(live, dv, npc)
        fire()

        lo8 = pl.multiple_of(lo, 8)
        rel8 = pl.multiple_of(rel, 8)
        pltpu.sync_copy(acc.at[pl.ds(rel8, RPT)], out.at[pl.ds(lo8, RPT)])
        if with_deg:
            pltpu.sync_copy(dgacc.at[pl.ds(rel8, RPT)],
                            dout.at[pl.ds(lo8, RPT)])

    return pl.kernel(body, out_type=out_type if with_deg else out_type[0],
                     mesh=mesh,
                     compiler_params=pltpu.CompilerParams(
                         needs_layout_passes=False),
                     scratch_types=scratch)


def _rowspec(k):
    return pl.BlockSpec((RB, k), lambda i: (i, 0))


def _fullspec(shape):
    return pl.BlockSpec(shape, lambda i: (0, 0))


def _recip_deg(dg):
    return 1.0 / jnp.maximum(dg[:, :1], 1.0)


def _layer1_body(p, dg, x, wl, bl, wr, ha, hb):
    recip = _recip_deg(dg[...])
    h = (jnp.dot(p[...], wl[...], preferred_element_type=jnp.float32) * recip
         + bl[...]
         + jnp.dot(x[...], wr[...], preferred_element_type=jnp.float32))
    h = jnp.maximum(h, 0.0)
    ha[...] = h[:, :D_IN]
    hb[...] = h[:, D_IN:]


def _layer1(p, dg, x, wl, bl, wr):
    return pl.pallas_call(
        _layer1_body,
        grid=(NPAD // RB,),
        in_specs=[_rowspec(D_IN), _rowspec(D_IN), _rowspec(D_IN),
                  _fullspec((D_IN, H)), _fullspec((1, H)),
                  _fullspec((D_IN, H))],
        out_specs=[_rowspec(D_IN), _rowspec(D_IN)],
        out_shape=[jax.ShapeDtypeStruct((NPAD, D_IN), jnp.float32)] * 2,
    )(p, dg, x, wl, bl, wr)


def _layer2_body(a, b, dg, ha, hb, wl, bl, wr, wl3, h2a, h2b, y):
    recip = _recip_deg(dg[...])
    wlv = wl[...]
    wrv = wr[...]
    t = (jnp.dot(a[...], wlv[:D_IN], preferred_element_type=jnp.float32)
         + jnp.dot(b[...], wlv[D_IN:], preferred_element_type=jnp.float32))
    h = (t * recip + bl[...]
         + jnp.dot(ha[...], wrv[:D_IN], preferred_element_type=jnp.float32)
         + jnp.dot(hb[...], wrv[D_IN:], preferred_element_type=jnp.float32))
    h = jnp.maximum(h, 0.0)
    h2a[...] = h[:, :D_IN]
    h2b[...] = h[:, D_IN:]
    y[...] = jnp.dot(h, wl3[...], preferred_element_type=jnp.float32)


def _layer2(a, b, dg, ha, hb, wl, bl, wr, wl3):
    return pl.pallas_call(
        _layer2_body,
        grid=(NPAD // RB,),
        in_specs=[_rowspec(D_IN)] * 3 + [_rowspec(D_IN)] * 2
                 + [_fullspec((H, H)), _fullspec((1, H)), _fullspec((H, H)),
                    _fullspec((H, CPAD))],
        out_specs=[_rowspec(D_IN), _rowspec(D_IN), _rowspec(CPAD)],
        out_shape=[jax.ShapeDtypeStruct((NPAD, D_IN), jnp.float32)] * 2
                  + [jax.ShapeDtypeStruct((NPAD, CPAD), jnp.float32)],
    )(a, b, dg, ha, hb, wl, bl, wr, wl3)


def _layer3_body(q, dg, ha, hb, wr, bl, out):
    recip = _recip_deg(dg[...])
    wrv = wr[...]
    z = (q[...] * recip + bl[...]
         + jnp.dot(ha[...], wrv[:D_IN], preferred_element_type=jnp.float32)
         + jnp.dot(hb[...], wrv[D_IN:], preferred_element_type=jnp.float32))
    m = jnp.max(z, axis=-1, keepdims=True)
    zs = z - m
    lse = jnp.log(jnp.sum(jnp.exp(zs), axis=-1, keepdims=True))
    out[...] = zs - lse


def _layer3(q, dg, ha, hb, wr, bl):
    return pl.pallas_call(
        _layer3_body,
        grid=(NPAD // RB,),
        in_specs=[_rowspec(CPAD), _rowspec(D_IN)]
                 + [_rowspec(D_IN)] * 2
                 + [_fullspec((H, CPAD)), _fullspec((1, CPAD))],
        out_specs=_rowspec(CPAD),
        out_shape=jax.ShapeDtypeStruct((NPAD, CPAD), jnp.float32),
    )(q, dg, ha, hb, wr, bl)


def kernel(x, edge_index, Wl1, bl1, Wr1, Wl2, bl2, Wr2, Wl3, bl3, Wr3):
    E = edge_index.shape[1]
    epad = -E % (CHUNK * IDXB)
    nchunk = (E + epad) // CHUNK
    src = jnp.concatenate(
        [edge_index[0], jnp.zeros((epad,), jnp.int32)]).reshape(-1, CHUNK)
    dst = jnp.concatenate(
        [edge_index[1], jnp.full((epad,), N, jnp.int32)]).reshape(-1, CHUNK)
    xp = jnp.pad(x, ((0, NPAD - N), (0, 0)))

    agg1, dg = _make_sc_agg(D_IN, nchunk, True)(xp, src, dst)
    h1a, h1b = _layer1(agg1, dg, xp, Wl1, bl1.reshape(1, H), Wr1)

    a2 = _make_sc_agg(D_IN, nchunk, False)(h1a, src, dst)
    b2 = _make_sc_agg(D_IN, nchunk, False)(h1b, src, dst)
    wl3p = jnp.pad(Wl3, ((0, 0), (0, CPAD - C)))
    h2a, h2b, y = _layer2(a2, b2, dg, h1a, h1b,
                          Wl2, bl2.reshape(1, H), Wr2, wl3p)

    q = _make_sc_agg(CPAD, nchunk, False)(y, src, dst)
    bl3p = jnp.concatenate(
        [bl3, jnp.full((CPAD - C,), -1e30, jnp.float32)]).reshape(1, CPAD)
    wr3p = jnp.pad(Wr3, ((0, 0), (0, CPAD - C)))
    z = _layer3(q, dg, h2a, h2b, wr3p, bl3p)
    return z[:N, :C]


# speedup vs baseline: 1.4653x; 1.4653x over previous
import functools

import jax
import jax.numpy as jnp
from jax import lax
from jax.experimental import pallas as pl
from jax.experimental.pallas import tpu as pltpu
from jax.experimental.pallas import tpu_sc as plsc

N = 10000
D_IN = 128
H = 256
C = 47
CPAD = 128

NC = 2
NS = 16
NW = NC * NS
LANES = 16

NPAD = 10240
RPT = NPAD // NW
CHUNK = 128
IDXB = 8
RB = 1280


SBLK = 32


def _make_sc_agg(nfeat, with_deg, nchunk):
    d = D_IN
    mesh = plsc.VectorSubcoreMesh(core_axis_name="c", subcore_axis_name="s")
    out_type = [jax.ShapeDtypeStruct((NPAD, d), jnp.float32)] * nfeat
    if with_deg:
        out_type.append(jax.ShapeDtypeStruct((NPAD, d), jnp.float32))
    npc = NPAD // NC
    scratch = [
        pltpu.VMEM((SBLK, CHUNK), jnp.int32),
        pltpu.VMEM((SBLK, CHUNK), jnp.int32),
        pltpu.VMEM((2 * CHUNK,), jnp.int32),
        pltpu.VMEM((2 * CHUNK,), jnp.int32),
        pltpu.VMEM((1, CHUNK), jnp.int32),
        pltpu.VMEM((1, CHUNK), jnp.int32),
        pltpu.VMEM((IDXB, d), jnp.float32),
    ]
    scratch += [pltpu.VMEM((CHUNK, d), jnp.float32)] * nfeat
    scratch += [pltpu.VMEM_SHARED((npc + 8, d), jnp.float32)] * nfeat
    scratch += [pltpu.SemaphoreType.DMA] * nfeat
    if with_deg:
        scratch += [
            pltpu.VMEM((CHUNK, d), jnp.float32),
            pltpu.VMEM_SHARED((npc + 8, d), jnp.float32),
        ]

    def body(*args):
        feats = args[:nfeat]
        src2d, dst2d = args[nfeat], args[nfeat + 1]
        outs = args[nfeat + 2:2 * nfeat + 2]
        k = 2 * nfeat + 2
        if with_deg:
            dout = args[k]
            k += 1
        sidx, didx, csrc, cdst, fsrc, fdst, zbuf = args[k:k + 7]
        k += 7
        rows = args[k:k + nfeat]
        accs = args[k + nfeat:k + 2 * nfeat]
        sems = args[k + 2 * nfeat:k + 3 * nfeat]
        k += 3 * nfeat
        if with_deg:
            ones, dgacc = args[k], args[k + 1]
        c = lax.axis_index("c")
        s = lax.axis_index("s")
        w = c * NS + s
        lo = w * RPT
        rel = s * RPT

        for i in range(IDXB):
            for j in range(d // LANES):
                zbuf[i, pl.ds(j * LANES, LANES)] = jnp.zeros(
                    (LANES,), jnp.float32)

        def zcp(t, _):
            for acc in accs:
                pltpu.sync_copy(zbuf, acc.at[pl.ds(rel + t * IDXB, IDXB)])
            if with_deg:
                pltpu.sync_copy(zbuf, dgacc.at[pl.ds(rel + t * IDXB, IDXB)])
            return 0
        lax.fori_loop(0, RPT // IDXB, zcp, 0)
        if with_deg:
            def orow(i, _):
                for j in range(d // LANES):
                    ones[i, pl.ds(j * LANES, LANES)] = jnp.ones(
                        (LANES,), jnp.float32)
                return 0
            lax.fori_loop(0, CHUNK, orow, 0)
        for v in range(2 * CHUNK // LANES):
            csrc[pl.ds(v * LANES, LANES)] = jnp.zeros((LANES,), jnp.int32)
            cdst[pl.ds(v * LANES, LANES)] = jnp.zeros((LANES,), jnp.int32)

        def fire():
            for v in range(CHUNK // LANES):
                fsrc[0, pl.ds(v * LANES, LANES)] = csrc[
                    pl.ds(v * LANES, LANES)]
                fdst[0, pl.ds(v * LANES, LANES)] = cdst[
                    pl.ds(v * LANES, LANES)]
            copies = [pltpu.async_copy(f.at[fsrc.at[0]], r, sm)
                      for f, r, sm in zip(feats, rows, sems)]
            for cp, r, acc in zip(copies, rows, accs):
                cp.wait()
                pltpu.sync_copy(r, acc.at[fdst.at[0]], add=True)
            if with_deg:
                pltpu.sync_copy(ones, dgacc.at[fdst.at[0]], add=True)

        def drain(cnt):
            def do(cnt):
                fire()
                for v in range(CHUNK // LANES):
                    csrc[pl.ds(v * LANES, LANES)] = csrc[
                        pl.ds(CHUNK + v * LANES, LANES)]
                    cdst[pl.ds(v * LANES, LANES)] = cdst[
                        pl.ds(CHUNK + v * LANES, LANES)]
                return cnt - CHUNK
            return lax.cond(cnt >= CHUNK, do, lambda q: q, cnt)

        def row(g, cnt):
            for u in range(CHUNK // LANES):
                dv = didx[g, pl.ds(u * LANES, LANES)]
                sv = sidx[g, pl.ds(u * LANES, LANES)]
                m = (dv >= lo) & (dv < lo + RPT)
                plsc.store_compressed(
                    csrc.at[pl.ds(cnt, LANES)], sv, mask=m)
                plsc.store_compressed(
                    cdst.at[pl.ds(cnt, LANES)], dv - (lo - rel), mask=m)
                cnt = cnt + jnp.max(
                    plsc.all_reduce_population_count(m))
            return drain(cnt)

        def blk(t, cnt):
            b = pl.multiple_of(t * SBLK, SBLK)
            pltpu.sync_copy(src2d.at[pl.ds(b, SBLK)], sidx)
            pltpu.sync_copy(dst2d.at[pl.ds(b, SBLK)], didx)
            return lax.fori_loop(0, SBLK, row, cnt)
        cnt = lax.fori_loop(0, nchunk // SBLK, blk, jnp.int32(0))

        pos = lax.iota(jnp.int32, LANES)
        for v in range(CHUNK // LANES):
            live = (pos + v * LANES) < cnt
            sv = csrc[pl.ds(v * LANES, LANES)]
            dv = cdst[pl.ds(v * LANES, LANES)]
            csrc[pl.ds(v * LANES, LANES)] = jnp.where(live, sv, 0)
            cdst[pl.ds(v * LANES, LANES)] = jnp.where(live, dv, npc)
        fire()

        lo8 = pl.multiple_of(lo, 8)
        rel8 = pl.multiple_of(rel, 8)
        for acc, out in zip(accs, outs):
            pltpu.sync_copy(acc.at[pl.ds(rel8, RPT)],
                            out.at[pl.ds(lo8, RPT)])
        if with_deg:
            pltpu.sync_copy(dgacc.at[pl.ds(rel8, RPT)],
                            dout.at[pl.ds(lo8, RPT)])

    return pl.kernel(body, out_type=out_type,
                     mesh=mesh,
                     compiler_params=pltpu.CompilerParams(
                         needs_layout_passes=False),
                     scratch_types=scratch)


def _rowspec(k):
    return pl.BlockSpec((RB, k), lambda i: (i, 0))


def _fullspec(shape):
    return pl.BlockSpec(shape, lambda i: (0, 0))


def _recip_deg(dg):
    return 1.0 / jnp.maximum(dg[:, :1], 1.0)


def _layer1_body(p, dg, x, wl, bl, wr, ha, hb):
    recip = _recip_deg(dg[...])
    h = (jnp.dot(p[...], wl[...], preferred_element_type=jnp.float32) * recip
         + bl[...]
         + jnp.dot(x[...], wr[...], preferred_element_type=jnp.float32))
    h = jnp.maximum(h, 0.0)
    ha[...] = h[:, :D_IN]
    hb[...] = h[:, D_IN:]


def _layer1(p, dg, x, wl, bl, wr):
    return pl.pallas_call(
        _layer1_body,
        grid=(NPAD // RB,),
        in_specs=[_rowspec(D_IN), _rowspec(D_IN), _rowspec(D_IN),
                  _fullspec((D_IN, H)), _fullspec((1, H)),
                  _fullspec((D_IN, H))],
        out_specs=[_rowspec(D_IN), _rowspec(D_IN)],
        out_shape=[jax.ShapeDtypeStruct((NPAD, D_IN), jnp.float32)] * 2,
    )(p, dg, x, wl, bl, wr)


def _layer2_body(a, b, dg, ha, hb, wl, bl, wr, wl3, h2a, h2b, y):
    recip = _recip_deg(dg[...])
    wlv = wl[...]
    wrv = wr[...]
    t = (jnp.dot(a[...], wlv[:D_IN], preferred_element_type=jnp.float32)
         + jnp.dot(b[...], wlv[D_IN:], preferred_element_type=jnp.float32))
    h = (t * recip + bl[...]
         + jnp.dot(ha[...], wrv[:D_IN], preferred_element_type=jnp.float32)
         + jnp.dot(hb[...], wrv[D_IN:], preferred_element_type=jnp.float32))
    h = jnp.maximum(h, 0.0)
    h2a[...] = h[:, :D_IN]
    h2b[...] = h[:, D_IN:]
    y[...] = jnp.dot(h, wl3[...], preferred_element_type=jnp.float32)


def _layer2(a, b, dg, ha, hb, wl, bl, wr, wl3):
    return pl.pallas_call(
        _layer2_body,
        grid=(NPAD // RB,),
        in_specs=[_rowspec(D_IN)] * 3 + [_rowspec(D_IN)] * 2
                 + [_fullspec((H, H)), _fullspec((1, H)), _fullspec((H, H)),
                    _fullspec((H, CPAD))],
        out_specs=[_rowspec(D_IN), _rowspec(D_IN), _rowspec(CPAD)],
        out_shape=[jax.ShapeDtypeStruct((NPAD, D_IN), jnp.float32)] * 2
                  + [jax.ShapeDtypeStruct((NPAD, CPAD), jnp.float32)],
    )(a, b, dg, ha, hb, wl, bl, wr, wl3)


def _layer3_body(q, dg, ha, hb, wr, bl, out):
    recip = _recip_deg(dg[...])
    wrv = wr[...]
    z = (q[...] * recip + bl[...]
         + jnp.dot(ha[...], wrv[:D_IN], preferred_element_type=jnp.float32)
         + jnp.dot(hb[...], wrv[D_IN:], preferred_element_type=jnp.float32))
    m = jnp.max(z, axis=-1, keepdims=True)
    zs = z - m
    lse = jnp.log(jnp.sum(jnp.exp(zs), axis=-1, keepdims=True))
    out[...] = zs - lse


def _layer3(q, dg, ha, hb, wr, bl):
    return pl.pallas_call(
        _layer3_body,
        grid=(NPAD // RB,),
        in_specs=[_rowspec(CPAD), _rowspec(D_IN)]
                 + [_rowspec(D_IN)] * 2
                 + [_fullspec((H, CPAD)), _fullspec((1, CPAD))],
        out_specs=_rowspec(CPAD),
        out_shape=jax.ShapeDtypeStruct((NPAD, CPAD), jnp.float32),
    )(q, dg, ha, hb, wr, bl)


def kernel(x, edge_index, Wl1, bl1, Wr1, Wl2, bl2, Wr2, Wl3, bl3, Wr3):
    E = edge_index.shape[1]
    epad = -E % (CHUNK * SBLK)
    nchunk = (E + epad) // CHUNK
    src = jnp.concatenate(
        [edge_index[0], jnp.zeros((epad,), jnp.int32)]).reshape(-1, CHUNK)
    dst = jnp.concatenate(
        [edge_index[1], jnp.full((epad,), N, jnp.int32)]).reshape(-1, CHUNK)
    xp = jnp.pad(x, ((0, NPAD - N), (0, 0)))

    agg1, dg = _make_sc_agg(1, True, nchunk)(xp, src, dst)
    h1a, h1b = _layer1(agg1, dg, xp, Wl1, bl1.reshape(1, H), Wr1)

    a2, b2 = _make_sc_agg(2, False, nchunk)(h1a, h1b, src, dst)
    wl3p = jnp.pad(Wl3, ((0, 0), (0, CPAD - C)))
    h2a, h2b, y = _layer2(a2, b2, dg, h1a, h1b,
                          Wl2, bl2.reshape(1, H), Wr2, wl3p)

    (q,) = _make_sc_agg(1, False, nchunk)(y, src, dst)
    bl3p = jnp.concatenate(
        [bl3, jnp.full((CPAD - C,), -1e30, jnp.float32)]).reshape(1, CPAD)
    wr3p = jnp.pad(Wr3, ((0, 0), (0, CPAD - C)))
    z = _layer3(q, dg, h2a, h2b, wr3p, bl3p)
    return z[:N, :C]
